# baseline (speedup 1.0000x reference)
import functools

import jax
import jax.numpy as jnp
from jax import lax
from jax.experimental import pallas as pl
from jax.experimental.pallas import tpu as pltpu
from jax.experimental.pallas import tpu_sc as plsc

N = 10000
E = 320000
D_IN, H1, H2 = 128, 64, 32

NC, NS = 2, 16
R_E = E // 128
RPT = 160
R_P = NS * RPT
NP = 10240
NPS = NP // NS
HW = 32
BLK = 1024
GRID = NP // BLK

F32 = jnp.float32


def _tc1_body(x_ref, w_ref, a_ref, wh_ref, s_ref):
    wh = jnp.dot(x_ref[...], w_ref[...], preferred_element_type=F32)
    wh_ref[0] = wh[:, :HW]
    wh_ref[1] = wh[:, HW:]
    a = a_ref[...]
    sa = jnp.dot(wh, a[:H1, :], preferred_element_type=F32)
    sb = jnp.dot(wh, a[H1:, :], preferred_element_type=F32)
    s_ref[...] = jnp.concatenate([sa, sb], axis=1)


def _tc1(x, W1, a1):
    return pl.pallas_call(
        _tc1_body,
        grid=(GRID,),
        in_specs=[
            pl.BlockSpec((BLK, D_IN), lambda i: (i, 0)),
            pl.BlockSpec((D_IN, H1), lambda i: (0, 0)),
            pl.BlockSpec((2 * H1, 1), lambda i: (0, 0)),
        ],
        out_specs=[
            pl.BlockSpec((NC, BLK, HW), lambda i: (0, i, 0)),
            pl.BlockSpec((BLK, 2), lambda i: (i, 0)),
        ],
        out_shape=[
            jax.ShapeDtypeStruct((NC, NP, HW), F32),
            jax.ShapeDtypeStruct((NP, 2), F32),
        ],
    )(x, W1, a1)


def _tc2_body(hp_ref, w2_ref, w3_ref, a2_ref, a3_ref, wh_ref, s_ref):
    h = jnp.maximum(jnp.concatenate([hp_ref[0], hp_ref[1]], axis=1), 0.0)
    w23 = jnp.concatenate([w2_ref[...], w3_ref[...]], axis=1)
    wh = jnp.dot(h, w23, preferred_element_type=F32)
    wh_ref[0] = wh[:, :H2]
    wh_ref[1] = wh[:, H2:]
    a2 = a2_ref[...]
    a3 = a3_ref[...]
    s2a = jnp.dot(wh[:, :H2], a2[:H2, :], preferred_element_type=F32)
    s2b = jnp.dot(wh[:, :H2], a2[H2:, :], preferred_element_type=F32)
    s3a = jnp.dot(wh[:, H2:], a3[:H2, :], preferred_element_type=F32)
    s3b = jnp.dot(wh[:, H2:], a3[H2:, :], preferred_element_type=F32)
    s_ref[...] = jnp.concatenate([s2a, s2b, s3a, s3b], axis=1)


def _tc2(hp1, W2, W3, a2, a3):
    return pl.pallas_call(
        _tc2_body,
        grid=(GRID,),
        in_specs=[
            pl.BlockSpec((NC, BLK, HW), lambda i: (0, i, 0)),
            pl.BlockSpec((H1, H2), lambda i: (0, 0)),
            pl.BlockSpec((H1, H2), lambda i: (0, 0)),
            pl.BlockSpec((2 * H2, 1), lambda i: (0, 0)),
            pl.BlockSpec((2 * H2, 1), lambda i: (0, 0)),
        ],
        out_specs=[
            pl.BlockSpec((NC, BLK, HW), lambda i: (0, i, 0)),
            pl.BlockSpec((BLK, 4), lambda i: (i, 0)),
        ],
        out_shape=[
            jax.ShapeDtypeStruct((NC, NP, HW), F32),
            jax.ShapeDtypeStruct((NP, 4), F32),
        ],
    )(hp1, W2, W3, a2, a3)


def _make_sc(stride):
    st = stride * N
    mesh = plsc.VectorSubcoreMesh(core_axis_name="c", subcore_axis_name="s")

    scratch = [
        pltpu.VMEM((st,), F32),
        pltpu.VMEM((RPT, 128), jnp.int32),
        pltpu.VMEM((RPT, 128), jnp.int32),
        pltpu.VMEM((NPS,), F32),
    ] + [pltpu.VMEM((1, 128), F32) for _ in range(4)] \
      + [pltpu.VMEM((128, HW), F32) for _ in range(4)] + [
        pltpu.VMEM_SHARED((NP,), F32),
        pltpu.VMEM_SHARED((NP, HW), F32),
    ] + [pltpu.SemaphoreType.DMA for _ in range(13)]

    @functools.partial(
        pl.kernel,
        out_type=jax.ShapeDtypeStruct((NC, NP, HW), F32),
        mesh=mesh,
        scratch_types=scratch,
        compiler_params=pltpu.CompilerParams(use_tc_tiling_on_sc=False,
                                             needs_layout_passes=False),
    )
    def sc_kernel(src_hbm, dst_hbm, s_hbm, wh_hbm, hp_out,
                  s_v, src_v, dst_v, den_v,
                  exr0, exr1, exr2, exr3, rows0, rows1, rows2, rows3,
                  den_sh, hp_sh,
                  gs0, gs1, gs2, gs3, ss0, ss1, ss2, ss3,
                  ds0, ds1, ds2, ds3, sem):
        exrs = (exr0, exr1, exr2, exr3)
        rows = (rows0, rows1, rows2, rows3)
        gss = (gs0, gs1, gs2, gs3)
        sss = (ss0, ss1, ss2, ss3)
        dss = (ds0, ds1, ds2, ds3)
        rowsA_v = rows0
        cid = lax.axis_index("c")
        sid = lax.axis_index("s")
        off = cid * (stride - 2)

        pltpu.sync_copy(s_hbm, s_v)
        pltpu.sync_copy(src_hbm.at[pl.ds(sid * RPT, RPT)], src_v)
        pltpu.sync_copy(dst_hbm.at[pl.ds(sid * RPT, RPT)], dst_v)

        def zrow(j, c):
            for q in range(HW // 16):
                rowsA_v[j, pl.ds(q * 16, 16)] = jnp.zeros((16,), F32)
            return c
        lax.fori_loop(0, 128, zrow, 0)

        def zden(k, c):
            den_v[pl.ds(k * 16, 16)] = jnp.zeros((16,), F32)
            return c
        lax.fori_loop(0, NPS // 16, zden, 0)
        pltpu.sync_copy(den_v, den_sh.at[pl.ds(sid * NPS, NPS)])
        for o in range(0, NPS, 128):
            pltpu.sync_copy(rowsA_v, hp_sh.at[pl.ds(sid * NPS + o, 128)])
        plsc.subcore_barrier()

        def compute_ex(r, exr_v):
            for k in range(8):
                sl = pl.ds(k * 16, 16)
                s16 = src_v[r, sl]
                d16 = dst_v[r, sl]
                g0 = plsc.load_gather(s_v, [s16 * stride + off])
                g1 = plsc.load_gather(s_v, [d16 * stride + (off + 1)])
                l0 = (g0 + g1) * 2.0
                l0 = jnp.maximum(l0, l0 * 0.01)
                exr_v[0, sl] = jnp.exp(jnp.minimum(l0, 80.0))

        def den_scatter(r, exr_v, dsem):
            pltpu.async_copy(exr_v.at[0], den_sh.at[src_v.at[r]], dsem,
                             add=True)

        def den_drain(dsem):
            pltpu.make_async_copy(exr0.at[0],
                                  den_sh.at[pl.ds(0, 128)], dsem).wait()

        def scale(rows_v, exr_v):
            for k in range(8):
                sl = pl.ds(k * 16, 16)
                e0 = exr_v[0, sl]
                for jj in range(16):
                    j = k * 16 + jj
                    a0 = e0[jj]
                    for q in range(HW // 16):
                        qs = pl.ds(q * 16, 16)
                        rows_v[j, qs] = rows_v[j, qs] * a0

        n_rows = jnp.clip(R_E - sid * RPT, 0, RPT)

        for i in range(4):
            pltpu.async_copy(wh_hbm.at[cid].at[dst_v.at[i]], rows[i], gss[i])
        for i in range(4):
            compute_ex(i, exrs[i])
            den_scatter(i, exrs[i], dss[i])

        n4 = n_rows // 4

        def body(r4, c):
            a = 4 * r4
            for i in range(4):
                pltpu.make_async_copy(wh_hbm.at[cid].at[dst_v.at[a + i]],
                                      rows[i], gss[i]).wait()
                scale(rows[i], exrs[i])
                pltpu.async_copy(rows[i], hp_sh.at[src_v.at[a + i]], sss[i],
                                 add=True)

            @pl.when(r4 + 1 < n4)
            def _prefetch():
                for i in range(4):
                    pltpu.make_async_copy(rows[i], hp_sh.at[src_v.at[a + i]],
                                          sss[i]).wait()
                    pltpu.async_copy(wh_hbm.at[cid].at[dst_v.at[a + 4 + i]],
                                     rows[i], gss[i])
                for i in range(4):
                    den_drain(dss[i])
                    compute_ex(a + 4 + i, exrs[i])
                    den_scatter(a + 4 + i, exrs[i], dss[i])
            return c

        lax.fori_loop(0, n4, body, 0)
        for i in range(4):
            pltpu.make_async_copy(rows[i], hp_sh.at[pl.ds(0, 128)],
                                  sss[i]).wait()
            den_drain(dss[i])
        plsc.subcore_barrier()

        pltpu.sync_copy(den_sh.at[pl.ds(sid * NPS, NPS)], den_v)
        base = sid * NPS

        def norm_body(b, c):
            rows_v = rowsA_v
            pltpu.async_copy(hp_sh.at[pl.ds(base + b * 128, 128)], rows_v,
                             sem).wait()
            for k in range(8):
                sl = pl.ds(b * 128 + k * 16, 16)
                d0 = den_v[sl]
                r0 = 1.0 / jnp.maximum(d0, 1e-30)
                for jj in range(16):
                    j = k * 16 + jj
                    a0 = r0[jj]
                    for q in range(HW // 16):
                        qs = pl.ds(q * 16, 16)
                        rows_v[j, qs] = rows_v[j, qs] * a0
            pltpu.sync_copy(rows_v,
                            hp_out.at[cid, pl.ds(base + b * 128, 128)])
            return c

        lax.fori_loop(0, NPS // 128, norm_body, 0)

    return sc_kernel


_sc1 = _make_sc(2)
_sc23 = _make_sc(4)


def kernel(x, edge_index, W1, a1, W2, a2, W3, a3):
    src2d = jnp.pad(edge_index[0].reshape(R_E, 128), ((0, R_P - R_E), (0, 0)))
    dst2d = jnp.pad(edge_index[1].reshape(R_E, 128), ((0, R_P - R_E), (0, 0)))
    x_p = jnp.pad(x, ((0, NP - N), (0, 0)))

    wh1, s1 = _tc1(x_p, W1, a1)
    hp1 = _sc1(src2d, dst2d, s1.reshape(-1)[:2 * N], wh1)
    wh23, s23 = _tc2(hp1, W2, W3, a2, a3)
    hp23 = _sc23(src2d, dst2d, s23.reshape(-1)[:4 * N], wh23)
    return (hp23[0, :N], hp23[0, :N], hp23[1, :N])


# speedup vs baseline: 1.0935x; 1.0935x over previous
import functools

import jax
import jax.numpy as jnp
from jax import lax
from jax.experimental import pallas as pl
from jax.experimental.pallas import tpu as pltpu
from jax.experimental.pallas import tpu_sc as plsc

N = 10000
E = 320000
D_IN, H1, H2 = 128, 64, 32

NC, NS = 2, 16
R_E = E // 128
RPT = 160
R_P = NS * RPT
NP = 10240
NPS = NP // NS
HW = 32
BLK = 1024
GRID = NP // BLK

F32 = jnp.float32


def _tc1_body(x_ref, w_ref, a_ref, wh_ref, s_ref):
    wh = jnp.dot(x_ref[...], w_ref[...], preferred_element_type=F32)
    wh_ref[0] = wh[:, :HW]
    wh_ref[1] = wh[:, HW:]
    a = a_ref[...]
    sa = jnp.dot(wh, a[:H1, :], preferred_element_type=F32)
    sb = jnp.dot(wh, a[H1:, :], preferred_element_type=F32)
    s_ref[...] = jnp.concatenate([sa, sb], axis=1)


def _tc1(x, W1, a1):
    return pl.pallas_call(
        _tc1_body,
        grid=(GRID,),
        in_specs=[
            pl.BlockSpec((BLK, D_IN), lambda i: (i, 0)),
            pl.BlockSpec((D_IN, H1), lambda i: (0, 0)),
            pl.BlockSpec((2 * H1, 1), lambda i: (0, 0)),
        ],
        out_specs=[
            pl.BlockSpec((NC, BLK, HW), lambda i: (0, i, 0)),
            pl.BlockSpec((BLK, 2), lambda i: (i, 0)),
        ],
        out_shape=[
            jax.ShapeDtypeStruct((NC, NP, HW), F32),
            jax.ShapeDtypeStruct((NP, 2), F32),
        ],
    )(x, W1, a1)


def _tc2_body(hp_ref, w2_ref, w3_ref, a2_ref, a3_ref, wh_ref, s_ref):
    h = jnp.maximum(jnp.concatenate([hp_ref[0], hp_ref[1]], axis=1), 0.0)
    w23 = jnp.concatenate([w2_ref[...], w3_ref[...]], axis=1)
    wh = jnp.dot(h, w23, preferred_element_type=F32)
    wh_ref[0] = wh[:, :H2]
    wh_ref[1] = wh[:, H2:]
    a2 = a2_ref[...]
    a3 = a3_ref[...]
    s2a = jnp.dot(wh[:, :H2], a2[:H2, :], preferred_element_type=F32)
    s2b = jnp.dot(wh[:, :H2], a2[H2:, :], preferred_element_type=F32)
    s3a = jnp.dot(wh[:, H2:], a3[:H2, :], preferred_element_type=F32)
    s3b = jnp.dot(wh[:, H2:], a3[H2:, :], preferred_element_type=F32)
    s_ref[...] = jnp.concatenate([s2a, s2b, s3a, s3b], axis=1)


def _tc2(hp1, W2, W3, a2, a3):
    return pl.pallas_call(
        _tc2_body,
        grid=(GRID,),
        in_specs=[
            pl.BlockSpec((NC, BLK, HW), lambda i: (0, i, 0)),
            pl.BlockSpec((H1, H2), lambda i: (0, 0)),
            pl.BlockSpec((H1, H2), lambda i: (0, 0)),
            pl.BlockSpec((2 * H2, 1), lambda i: (0, 0)),
            pl.BlockSpec((2 * H2, 1), lambda i: (0, 0)),
        ],
        out_specs=[
            pl.BlockSpec((NC, BLK, HW), lambda i: (0, i, 0)),
            pl.BlockSpec((BLK, 4), lambda i: (i, 0)),
        ],
        out_shape=[
            jax.ShapeDtypeStruct((NC, NP, HW), F32),
            jax.ShapeDtypeStruct((NP, 4), F32),
        ],
    )(hp1, W2, W3, a2, a3)


def _make_sc(stride):
    st = stride * N
    mesh = plsc.VectorSubcoreMesh(core_axis_name="c", subcore_axis_name="s")

    scratch = [
        pltpu.VMEM((st,), F32),
        pltpu.VMEM((RPT, 128), jnp.int32),
        pltpu.VMEM((RPT, 128), jnp.int32),
        pltpu.VMEM((NPS,), F32),
    ] + [pltpu.VMEM((1, 128), F32) for _ in range(2)] \
      + [pltpu.VMEM((128, HW), F32) for _ in range(2)] + [
        pltpu.VMEM_SHARED((NP,), F32),
        pltpu.VMEM_SHARED((NP, HW), F32),
    ] + [pltpu.SemaphoreType.DMA for _ in range(7)]

    @functools.partial(
        pl.kernel,
        out_type=jax.ShapeDtypeStruct((NC, NP, HW), F32),
        mesh=mesh,
        scratch_types=scratch,
        compiler_params=pltpu.CompilerParams(use_tc_tiling_on_sc=False,
                                             needs_layout_passes=False),
    )
    def sc_kernel(src_hbm, dst_hbm, s_hbm, wh_hbm, hp_out,
                  s_v, src_v, dst_v, den_v,
                  exr0, exr1, rows0, rows1,
                  den_sh, hp_sh,
                  gs0, gs1, ss0, ss1, ds0, ds1, sem):
        exrs = (exr0, exr1)
        rows = (rows0, rows1)
        gss = (gs0, gs1)
        sss = (ss0, ss1)
        dss = (ds0, ds1)
        rowsA_v = rows0
        cid = lax.axis_index("c")
        sid = lax.axis_index("s")
        off = cid * (stride - 2)

        pltpu.sync_copy(s_hbm, s_v)
        pltpu.sync_copy(src_hbm.at[pl.ds(sid * RPT, RPT)], src_v)
        pltpu.sync_copy(dst_hbm.at[pl.ds(sid * RPT, RPT)], dst_v)

        def zrow(j, c):
            for q in range(HW // 16):
                rowsA_v[j, pl.ds(q * 16, 16)] = jnp.zeros((16,), F32)
            return c
        lax.fori_loop(0, 128, zrow, 0)

        def zden(k, c):
            den_v[pl.ds(k * 16, 16)] = jnp.zeros((16,), F32)
            return c
        lax.fori_loop(0, NPS // 16, zden, 0)
        pltpu.sync_copy(den_v, den_sh.at[pl.ds(sid * NPS, NPS)])
        for o in range(0, NPS, 128):
            pltpu.sync_copy(rowsA_v, hp_sh.at[pl.ds(sid * NPS + o, 128)])
        plsc.subcore_barrier()

        def compute_ex(r, exr_v):
            for k in range(8):
                sl = pl.ds(k * 16, 16)
                s16 = src_v[r, sl]
                d16 = dst_v[r, sl]
                g0 = plsc.load_gather(s_v, [s16 * stride + off])
                g1 = plsc.load_gather(s_v, [d16 * stride + (off + 1)])
                l0 = (g0 + g1) * 2.0
                l0 = jnp.maximum(l0, l0 * 0.01)
                exr_v[0, sl] = jnp.exp(jnp.minimum(l0, 80.0))

        def den_scatter(r, exr_v, dsem):
            pltpu.async_copy(exr_v.at[0], den_sh.at[src_v.at[r]], dsem,
                             add=True)

        def den_drain(dsem):
            pltpu.make_async_copy(exr0.at[0],
                                  den_sh.at[pl.ds(0, 128)], dsem).wait()

        def scale(rows_v, exr_v):
            for k in range(8):
                sl = pl.ds(k * 16, 16)
                e0 = exr_v[0, sl]
                for jj in range(16):
                    j = k * 16 + jj
                    a0 = e0[jj]
                    for q in range(HW // 16):
                        qs = pl.ds(q * 16, 16)
                        rows_v[j, qs] = rows_v[j, qs] * a0

        n_rows = jnp.clip(R_E - sid * RPT, 0, RPT)

        for i in range(2):
            pltpu.async_copy(wh_hbm.at[cid].at[dst_v.at[i]], rows[i], gss[i])
        for i in range(2):
            compute_ex(i, exrs[i])
            den_scatter(i, exrs[i], dss[i])

        n2 = n_rows // 2

        def body(r2, c):
            a = 2 * r2
            for i in range(2):
                pltpu.make_async_copy(wh_hbm.at[cid].at[dst_v.at[a + i]],
                                      rows[i], gss[i]).wait()
                scale(rows[i], exrs[i])
                pltpu.async_copy(rows[i], hp_sh.at[src_v.at[a + i]], sss[i],
                                 add=True)

            @pl.when(r2 + 1 < n2)
            def _prefetch():
                for i in range(2):
                    pltpu.make_async_copy(rows[i], hp_sh.at[src_v.at[a + i]],
                                          sss[i]).wait()
                    pltpu.async_copy(wh_hbm.at[cid].at[dst_v.at[a + 2 + i]],
                                     rows[i], gss[i])
                for i in range(2):
                    den_drain(dss[i])
                    compute_ex(a + 2 + i, exrs[i])
                    den_scatter(a + 2 + i, exrs[i], dss[i])
            return c

        lax.fori_loop(0, n2, body, 0)
        for i in range(2):
            pltpu.make_async_copy(rows[i], hp_sh.at[pl.ds(0, 128)],
                                  sss[i]).wait()
            den_drain(dss[i])
        plsc.subcore_barrier()

        pltpu.sync_copy(den_sh.at[pl.ds(sid * NPS, NPS)], den_v)
        base = sid * NPS

        def norm_body(b, c):
            rows_v = rowsA_v
            pltpu.async_copy(hp_sh.at[pl.ds(base + b * 128, 128)], rows_v,
                             sem).wait()
            for k in range(8):
                sl = pl.ds(b * 128 + k * 16, 16)
                d0 = den_v[sl]
                r0 = 1.0 / jnp.maximum(d0, 1e-30)
                for jj in range(16):
                    j = k * 16 + jj
                    a0 = r0[jj]
                    for q in range(HW // 16):
                        qs = pl.ds(q * 16, 16)
                        rows_v[j, qs] = rows_v[j, qs] * a0
            pltpu.sync_copy(rows_v,
                            hp_out.at[cid, pl.ds(base + b * 128, 128)])
            return c

        lax.fori_loop(0, NPS // 128, norm_body, 0)

    return sc_kernel


_sc1 = _make_sc(2)
_sc23 = _make_sc(4)


def kernel(x, edge_index, W1, a1, W2, a2, W3, a3):
    src2d = jnp.pad(edge_index[0].reshape(R_E, 128), ((0, R_P - R_E), (0, 0)))
    dst2d = jnp.pad(edge_index[1].reshape(R_E, 128), ((0, R_P - R_E), (0, 0)))
    x_p = jnp.pad(x, ((0, NP - N), (0, 0)))

    wh1, s1 = _tc1(x_p, W1, a1)
    hp1 = _sc1(src2d, dst2d, s1.reshape(-1)[:2 * N], wh1)
    wh23, s23 = _tc2(hp1, W2, W3, a2, a3)
    hp23 = _sc23(src2d, dst2d, s23.reshape(-1)[:4 * N], wh23)
    return (hp23[0, :N], hp23[0, :N], hp23[1, :N])


# speedup vs baseline: 1.1058x; 1.0113x over previous
import functools

import jax
import jax.numpy as jnp
from jax import lax
from jax.experimental import pallas as pl
from jax.experimental.pallas import tpu as pltpu
from jax.experimental.pallas import tpu_sc as plsc

N = 10000
E = 320000
D_IN, H1, H2 = 128, 64, 32

NC, NS = 2, 16
R_E = E // 128
RPT = 160
R_P = NS * RPT
NP = 10240
NPS = NP // NS
HW = 32
BLK = 1024
GRID = NP // BLK

F32 = jnp.float32


def _tc1_body(x_ref, w_ref, a_ref, wh_ref, s_ref):
    wh = jnp.dot(x_ref[...], w_ref[...], preferred_element_type=F32)
    wh_ref[0] = wh[:, :HW]
    wh_ref[1] = wh[:, HW:]
    a = a_ref[...]
    sa = jnp.dot(wh, a[:H1, :], preferred_element_type=F32)
    sb = jnp.dot(wh, a[H1:, :], preferred_element_type=F32)
    s_ref[...] = jnp.concatenate([sa, sb], axis=1)


def _tc1(x, W1, a1):
    return pl.pallas_call(
        _tc1_body,
        grid=(GRID,),
        in_specs=[
            pl.BlockSpec((BLK, D_IN), lambda i: (i, 0)),
            pl.BlockSpec((D_IN, H1), lambda i: (0, 0)),
            pl.BlockSpec((2 * H1, 1), lambda i: (0, 0)),
        ],
        out_specs=[
            pl.BlockSpec((NC, BLK, HW), lambda i: (0, i, 0)),
            pl.BlockSpec((BLK, 2), lambda i: (i, 0)),
        ],
        out_shape=[
            jax.ShapeDtypeStruct((NC, NP, HW), F32),
            jax.ShapeDtypeStruct((NP, 2), F32),
        ],
    )(x, W1, a1)


def _tc2_body(hp_ref, w2_ref, w3_ref, a2_ref, a3_ref, wh_ref, s_ref):
    h = jnp.maximum(jnp.concatenate([hp_ref[0], hp_ref[1]], axis=1), 0.0)
    w23 = jnp.concatenate([w2_ref[...], w3_ref[...]], axis=1)
    wh = jnp.dot(h, w23, preferred_element_type=F32)
    wh_ref[0] = wh[:, :H2]
    wh_ref[1] = wh[:, H2:]
    a2 = a2_ref[...]
    a3 = a3_ref[...]
    s2a = jnp.dot(wh[:, :H2], a2[:H2, :], preferred_element_type=F32)
    s2b = jnp.dot(wh[:, :H2], a2[H2:, :], preferred_element_type=F32)
    s3a = jnp.dot(wh[:, H2:], a3[:H2, :], preferred_element_type=F32)
    s3b = jnp.dot(wh[:, H2:], a3[H2:, :], preferred_element_type=F32)
    s_ref[...] = jnp.concatenate([s2a, s2b, s3a, s3b], axis=1)


def _tc2(hp1, W2, W3, a2, a3):
    return pl.pallas_call(
        _tc2_body,
        grid=(GRID,),
        in_specs=[
            pl.BlockSpec((NC, BLK, HW), lambda i: (0, i, 0)),
            pl.BlockSpec((H1, H2), lambda i: (0, 0)),
            pl.BlockSpec((H1, H2), lambda i: (0, 0)),
            pl.BlockSpec((2 * H2, 1), lambda i: (0, 0)),
            pl.BlockSpec((2 * H2, 1), lambda i: (0, 0)),
        ],
        out_specs=[
            pl.BlockSpec((NC, BLK, HW), lambda i: (0, i, 0)),
            pl.BlockSpec((BLK, 4), lambda i: (i, 0)),
        ],
        out_shape=[
            jax.ShapeDtypeStruct((NC, NP, HW), F32),
            jax.ShapeDtypeStruct((NP, 4), F32),
        ],
    )(hp1, W2, W3, a2, a3)


def _make_sc(stride):
    st = stride * N
    mesh = plsc.VectorSubcoreMesh(core_axis_name="c", subcore_axis_name="s")

    scratch = [
        pltpu.VMEM((st,), F32),
        pltpu.VMEM((RPT, 128), jnp.int32),
        pltpu.VMEM((RPT, 128), jnp.int32),
        pltpu.VMEM((NPS,), F32),
    ] + [pltpu.VMEM((1, 128), F32) for _ in range(2)] \
      + [pltpu.VMEM((128, HW), F32) for _ in range(2)] + [
        pltpu.VMEM_SHARED((NP,), F32),
        pltpu.VMEM_SHARED((NP, HW), F32),
    ] + [pltpu.SemaphoreType.DMA for _ in range(8)]

    @functools.partial(
        pl.kernel,
        out_type=jax.ShapeDtypeStruct((NC, NP, HW), F32),
        mesh=mesh,
        scratch_types=scratch,
        compiler_params=pltpu.CompilerParams(use_tc_tiling_on_sc=False,
                                             needs_layout_passes=False),
    )
    def sc_kernel(src_hbm, dst_hbm, s_hbm, wh_hbm, hp_out,
                  s_v, src_v, dst_v, den_v,
                  exr0, exr1, rows0, rows1,
                  den_sh, hp_sh,
                  gs0, gs1, ss0, ss1, ds0, ds1, sem, wsem):
        exrs = (exr0, exr1)
        rows = (rows0, rows1)
        gss = (gs0, gs1)
        sss = (ss0, ss1)
        dss = (ds0, ds1)
        rowsA_v = rows0
        cid = lax.axis_index("c")
        sid = lax.axis_index("s")
        off = cid * (stride - 2)

        qn = st // 4
        for k in range(4):
            q = (sid + k) % 4
            pltpu.sync_copy(s_hbm.at[pl.ds(q * qn, qn)],
                            s_v.at[pl.ds(q * qn, qn)])
        pltpu.sync_copy(src_hbm.at[pl.ds(sid * RPT, RPT)], src_v)
        pltpu.sync_copy(dst_hbm.at[pl.ds(sid * RPT, RPT)], dst_v)

        def zrow(j, c):
            for q in range(HW // 16):
                rowsA_v[j, pl.ds(q * 16, 16)] = jnp.zeros((16,), F32)
            return c
        lax.fori_loop(0, 128, zrow, 0)

        def zden(k, c):
            den_v[pl.ds(k * 16, 16)] = jnp.zeros((16,), F32)
            return c
        lax.fori_loop(0, NPS // 16, zden, 0)
        pltpu.sync_copy(den_v, den_sh.at[pl.ds(sid * NPS, NPS)])
        for o in range(0, NPS, 128):
            pltpu.sync_copy(rowsA_v, hp_sh.at[pl.ds(sid * NPS + o, 128)])
        plsc.subcore_barrier()

        def compute_ex(r, exr_v):
            for k in range(8):
                sl = pl.ds(k * 16, 16)
                s16 = src_v[r, sl]
                d16 = dst_v[r, sl]
                g0 = plsc.load_gather(s_v, [s16 * stride + off])
                g1 = plsc.load_gather(s_v, [d16 * stride + (off + 1)])
                l0 = (g0 + g1) * 2.0
                l0 = jnp.maximum(l0, l0 * 0.01)
                exr_v[0, sl] = jnp.exp(jnp.minimum(l0, 80.0))

        def den_scatter(r, exr_v, dsem):
            pltpu.async_copy(exr_v.at[0], den_sh.at[src_v.at[r]], dsem,
                             add=True)

        def den_drain(dsem):
            pltpu.make_async_copy(exr0.at[0],
                                  den_sh.at[pl.ds(0, 128)], dsem).wait()

        def scale(rows_v, exr_v):
            for k in range(8):
                sl = pl.ds(k * 16, 16)
                e0 = exr_v[0, sl]
                for jj in range(16):
                    j = k * 16 + jj
                    a0 = e0[jj]
                    for q in range(HW // 16):
                        qs = pl.ds(q * 16, 16)
                        rows_v[j, qs] = rows_v[j, qs] * a0

        n_rows = jnp.clip(R_E - sid * RPT, 0, RPT)

        for i in range(2):
            pltpu.async_copy(wh_hbm.at[cid].at[dst_v.at[i]], rows[i], gss[i])
        for i in range(2):
            compute_ex(i, exrs[i])
            den_scatter(i, exrs[i], dss[i])

        n2 = n_rows // 2

        def body(r2, c):
            a = 2 * r2
            for i in range(2):
                pltpu.make_async_copy(wh_hbm.at[cid].at[dst_v.at[a + i]],
                                      rows[i], gss[i]).wait()
                scale(rows[i], exrs[i])
                pltpu.async_copy(rows[i], hp_sh.at[src_v.at[a + i]], sss[i],
                                 add=True)

            @pl.when(r2 + 1 < n2)
            def _prefetch():
                for i in range(2):
                    pltpu.make_async_copy(rows[i], hp_sh.at[src_v.at[a + i]],
                                          sss[i]).wait()
                    pltpu.async_copy(wh_hbm.at[cid].at[dst_v.at[a + 2 + i]],
                                     rows[i], gss[i])
                for i in range(2):
                    den_drain(dss[i])
                    compute_ex(a + 2 + i, exrs[i])
                    den_scatter(a + 2 + i, exrs[i], dss[i])
            return c

        lax.fori_loop(0, n2, body, 0)
        for i in range(2):
            pltpu.make_async_copy(rows[i], hp_sh.at[pl.ds(0, 128)],
                                  sss[i]).wait()
            den_drain(dss[i])
        plsc.subcore_barrier()

        pltpu.sync_copy(den_sh.at[pl.ds(sid * NPS, NPS)], den_v)
        base = sid * NPS

        def norm_body(b, c):
            rows_v = rowsA_v

            @pl.when(b > 0)
            def _wait_prev():
                pltpu.make_async_copy(
                    rows_v, hp_out.at[cid, pl.ds(base, 128)], wsem).wait()
            pltpu.async_copy(hp_sh.at[pl.ds(base + b * 128, 128)], rows_v,
                             sem).wait()
            for k in range(8):
                sl = pl.ds(b * 128 + k * 16, 16)
                d0 = den_v[sl]
                r0 = 1.0 / jnp.maximum(d0, 1e-30)
                for jj in range(16):
                    j = k * 16 + jj
                    a0 = r0[jj]
                    for q in range(HW // 16):
                        qs = pl.ds(q * 16, 16)
                        rows_v[j, qs] = rows_v[j, qs] * a0
            pltpu.async_copy(rows_v,
                             hp_out.at[cid, pl.ds(base + b * 128, 128)],
                             wsem)
            return c

        lax.fori_loop(0, NPS // 128, norm_body, 0)
        pltpu.make_async_copy(rowsA_v, hp_out.at[cid, pl.ds(base, 128)],
                              wsem).wait()

    return sc_kernel


_sc1 = _make_sc(2)
_sc23 = _make_sc(4)


def kernel(x, edge_index, W1, a1, W2, a2, W3, a3):
    src2d = jnp.pad(edge_index[0].reshape(R_E, 128), ((0, R_P - R_E), (0, 0)))
    dst2d = jnp.pad(edge_index[1].reshape(R_E, 128), ((0, R_P - R_E), (0, 0)))

    wh1, s1 = _tc1(x, W1, a1)
    hp1 = _sc1(src2d, dst2d, s1.reshape(-1)[:2 * N], wh1)
    wh23, s23 = _tc2(hp1, W2, W3, a2, a3)
    hp23 = _sc23(src2d, dst2d, s23.reshape(-1)[:4 * N], wh23)
    return (hp23[0, :N], hp23[0, :N], hp23[1, :N])
